# Initial kernel scaffold; baseline (speedup 1.0000x reference)
import jax
import jax.numpy as jnp
from jax.experimental import pallas as pl


def kernel(x, edge_index, batch, W1, b1, W2, b2):
    raise NotImplementedError("write your pallas kernel here")


# speedup vs baseline: 16.1179x; 16.1179x over previous
import functools

import jax
import jax.numpy as jnp
from jax import lax
from jax.experimental import pallas as pl
from jax.experimental.pallas import tpu as pltpu
from jax.experimental.pallas import tpu_sc as plsc

_NC = 2
_NS = 16
_NW = _NC * _NS
_CH = 128


def _tile_rows(N):
    per = (-(-N // _NS) + 15) // 16 * 16
    last = N - per * (_NS - 1)
    assert last > 0 and last % 16 == 0
    return per, last


def _degree_call(N, E):
    n_chunks = E // _CH
    assert n_chunks * _CH == E
    base, rem = divmod(n_chunks, _NW)
    per, last = _tile_rows(N)
    Np = (N + 15) // 16 * 16
    mesh = plsc.VectorSubcoreMesh(core_axis_name="c", subcore_axis_name="s")

    @functools.partial(
        pl.kernel,
        out_type=jax.ShapeDtypeStruct((2 * N,), jnp.float32),
        mesh=mesh,
        scratch_types=[
            pltpu.VMEM((_CH,), jnp.int32),
            pltpu.VMEM((N,), jnp.float32),
            pltpu.VMEM((per,), jnp.float32),
            pltpu.VMEM((per,), jnp.float32),
            pltpu.VMEM_SHARED((_NS * Np,), jnp.float32),
        ],
        compiler_params=pltpu.CompilerParams(needs_layout_passes=False),
    )
    def deg_kernel(dst_hbm, zero_hbm, out_hbm, didx, cnt, acc_s, tmp, acc):
        c = lax.axis_index("c")
        s = lax.axis_index("s")
        wid = s * _NC + c
        pltpu.sync_copy(zero_hbm, cnt)
        nmine = jnp.where(wid < rem, base + 1, base)
        ones16 = jnp.ones((16,), jnp.float32)

        def body(k, carry):
            off = (wid + k * _NW) * _CH
            pltpu.sync_copy(dst_hbm.at[pl.ds(off, _CH)], didx)
            for j in range(_CH // 16):
                plsc.addupdate_scatter(cnt, [didx[pl.ds(j * 16, 16)]], ones16)
            return carry

        lax.fori_loop(0, nmine, body, 0)
        pltpu.sync_copy(cnt, acc.at[pl.ds(s * Np, N)])
        plsc.subcore_barrier()
        r0 = s * per

        def combine(nr):
            pltpu.sync_copy(acc.at[pl.ds(r0, nr)], acc_s.at[pl.ds(0, nr)])

            def add_one(t, carry):
                pltpu.sync_copy(acc.at[pl.ds(t * Np + r0, nr)], tmp.at[pl.ds(0, nr)])
                for j in range(nr // 16):
                    sl = pl.ds(j * 16, 16)
                    acc_s[sl] = acc_s[sl] + tmp[sl]
                return carry

            lax.fori_loop(1, _NS, add_one, 0)
            pltpu.sync_copy(acc_s.at[pl.ds(0, nr)],
                            out_hbm.at[pl.ds(c * N + r0, nr)])

        @pl.when(s < _NS - 1)
        def _():
            combine(per)

        @pl.when(s == _NS - 1)
        def _():
            combine(last)

    return deg_kernel


def _aggregate_call(N, E, D):
    n_chunks = E // _CH
    base, rem = divmod(n_chunks, _NW)
    per, last = _tile_rows(N)
    mesh = plsc.VectorSubcoreMesh(core_axis_name="c", subcore_axis_name="s")

    @functools.partial(
        pl.kernel,
        out_type=jax.ShapeDtypeStruct((2 * N, D), jnp.float32),
        mesh=mesh,
        scratch_types=[
            pltpu.VMEM((_CH,), jnp.int32),
            pltpu.VMEM((_CH,), jnp.int32),
            pltpu.VMEM((_CH, D), jnp.float32),
            pltpu.VMEM_SHARED((N, D), jnp.float32),
            pltpu.SemaphoreType.DMA,
        ],
    )
    def agg_kernel(hp_hbm, src_hbm, dst_hbm, zero_hbm, out_hbm,
                   sidx, didx, rows, acc, sem):
        c = lax.axis_index("c")
        s = lax.axis_index("s")
        wid = s * _NC + c

        @pl.when(s < _NS - 1)
        def _():
            pltpu.sync_copy(zero_hbm.at[pl.ds(s * per, per)],
                            acc.at[pl.ds(s * per, per)])

        @pl.when(s == _NS - 1)
        def _():
            pltpu.sync_copy(zero_hbm.at[pl.ds(per * (_NS - 1), last)],
                            acc.at[pl.ds(per * (_NS - 1), last)])

        plsc.subcore_barrier()
        nmine = jnp.where(wid < rem, base + 1, base)

        def body(k, carry):
            off = (wid + k * _NW) * _CH
            pltpu.sync_copy(src_hbm.at[pl.ds(off, _CH)], sidx)
            pltpu.sync_copy(dst_hbm.at[pl.ds(off, _CH)], didx)
            pltpu.async_copy(hp_hbm.at[sidx], rows, sem).wait()
            pltpu.sync_copy(rows, acc.at[didx], add=True)
            return carry

        lax.fori_loop(0, nmine, body, 0)
        plsc.subcore_barrier()
        out_base = c * N

        @pl.when(s < _NS - 1)
        def _():
            pltpu.sync_copy(acc.at[pl.ds(s * per, per)],
                            out_hbm.at[pl.ds(out_base + s * per, per)])

        @pl.when(s == _NS - 1)
        def _():
            pltpu.sync_copy(acc.at[pl.ds(per * (_NS - 1), last)],
                            out_hbm.at[pl.ds(out_base + per * (_NS - 1), last)])

    return agg_kernel


def _matmul_scale_call(N, DI, DH, BN=1000):
    def body(x_ref, w_ref, d0_ref, d1_ref, hp_ref, dinv_ref):
        deg = d0_ref[...] + d1_ref[...] + 1.0
        dinv = lax.rsqrt(deg)
        h = jnp.dot(x_ref[...], w_ref[...], preferred_element_type=jnp.float32)
        hp_ref[...] = h * dinv
        dinv_ref[...] = dinv

    return pl.pallas_call(
        body,
        grid=(N // BN,),
        in_specs=[
            pl.BlockSpec((BN, DI), lambda i: (i, 0)),
            pl.BlockSpec((DI, DH), lambda i: (0, 0)),
            pl.BlockSpec((BN, 1), lambda i: (i, 0)),
            pl.BlockSpec((BN, 1), lambda i: (i, 0)),
        ],
        out_specs=[
            pl.BlockSpec((BN, DH), lambda i: (i, 0)),
            pl.BlockSpec((BN, 1), lambda i: (i, 0)),
        ],
        out_shape=[
            jax.ShapeDtypeStruct((N, DH), jnp.float32),
            jax.ShapeDtypeStruct((N, 1), jnp.float32),
        ],
    )


def _layer2_call(N, DH, DO, BN=1000):
    def body(a0_ref, a1_ref, hp_ref, dinv_ref, b1_ref, w2_ref, out_ref):
        agg = a0_ref[...] + a1_ref[...] + hp_ref[...]
        h1 = jnp.maximum(dinv_ref[...] * agg + b1_ref[...], 0.0)
        out_ref[...] = dinv_ref[...] * jnp.dot(
            h1, w2_ref[...], preferred_element_type=jnp.float32)

    return pl.pallas_call(
        body,
        grid=(N // BN,),
        in_specs=[
            pl.BlockSpec((BN, DH), lambda i: (i, 0)),
            pl.BlockSpec((BN, DH), lambda i: (i, 0)),
            pl.BlockSpec((BN, DH), lambda i: (i, 0)),
            pl.BlockSpec((BN, 1), lambda i: (i, 0)),
            pl.BlockSpec((1, DH), lambda i: (0, 0)),
            pl.BlockSpec((DH, DO), lambda i: (0, 0)),
        ],
        out_specs=pl.BlockSpec((BN, DO), lambda i: (i, 0)),
        out_shape=jax.ShapeDtypeStruct((N, DO), jnp.float32),
    )


def _pool_call(N, DO, G, BN=1000):
    nblk = N // BN

    def body(a0_ref, a1_ref, hp_ref, dinv_ref, b2_ref, batch_ref, out_ref,
             sums, cnt):
        i = pl.program_id(0)

        @pl.when(i == 0)
        def _():
            sums[...] = jnp.zeros_like(sums)
            cnt[...] = jnp.zeros_like(cnt)

        agg = a0_ref[...] + a1_ref[...] + hp_ref[...]
        h2 = dinv_ref[...] * agg + b2_ref[...]
        ind = (batch_ref[...] == lax.broadcasted_iota(jnp.int32, (1, G), 1)
               ).astype(jnp.float32)
        sums[...] += lax.dot_general(
            ind, h2, (((0,), (0,)), ((), ())),
            preferred_element_type=jnp.float32)
        cnt[...] += lax.dot_general(
            ind, jnp.ones((BN, 1), jnp.float32), (((0,), (0,)), ((), ())),
            preferred_element_type=jnp.float32)
        out_ref[...] = sums[...] / jnp.maximum(cnt[...], 1.0)

    return pl.pallas_call(
        body,
        grid=(nblk,),
        in_specs=[
            pl.BlockSpec((BN, DO), lambda i: (i, 0)),
            pl.BlockSpec((BN, DO), lambda i: (i, 0)),
            pl.BlockSpec((BN, DO), lambda i: (i, 0)),
            pl.BlockSpec((BN, 1), lambda i: (i, 0)),
            pl.BlockSpec((1, DO), lambda i: (0, 0)),
            pl.BlockSpec((BN, 1), lambda i: (i, 0)),
        ],
        out_specs=pl.BlockSpec((G, DO), lambda i: (0, 0)),
        out_shape=jax.ShapeDtypeStruct((G, DO), jnp.float32),
        scratch_shapes=[
            pltpu.VMEM((G, DO), jnp.float32),
            pltpu.VMEM((G, 1), jnp.float32),
        ],
    )


def kernel(x, edge_index, batch, W1, b1, W2, b2):
    N, DI = x.shape
    DH = W1.shape[1]
    DO = W2.shape[1]
    E = edge_index.shape[1]
    G = 16

    src = edge_index[0]
    dst = edge_index[1]
    zeros_d = jnp.zeros((N,), jnp.float32)
    zeros_h = jnp.zeros((N, max(DH, DO)), jnp.float32)

    deg2 = _degree_call(N, E)(dst, zeros_d)
    d0 = deg2[:N].reshape(N, 1)
    d1 = deg2[N:].reshape(N, 1)
    h1p, dinv = _matmul_scale_call(N, DI, DH)(x, W1, d0, d1)
    agg1 = _aggregate_call(N, E, DH)(h1p, src, dst, zeros_h[:, :DH])
    h2p = _layer2_call(N, DH, DO)(
        agg1[:N], agg1[N:], h1p, dinv, b1.reshape(1, -1), W2)
    agg2 = _aggregate_call(N, E, DO)(h2p, src, dst, zeros_h[:, :DO])
    out = _pool_call(N, DO, G)(
        agg2[:N], agg2[N:], h2p, dinv, b2.reshape(1, -1), batch.reshape(-1, 1))
    return out
